# baseline (speedup 1.0000x reference)
import functools

import jax
import jax.numpy as jnp
from jax import lax
from jax.experimental import pallas as pl
from jax.experimental.pallas import tpu as pltpu
from jax.experimental.pallas import tpu_sc as plsc

_TILE_N = 128
_W = 128
_NC, _NS = 2, 16
_NW = _NC * _NS


def _dist_argmin_body(scale, z_ref, cbt2_ref, b2_ref, idx_ref, loss_ref):
    i = pl.program_id(0)
    t = z_ref.shape[0]
    k = cbt2_ref.shape[1]
    z = z_ref[...]
    a2 = jnp.sum(z * z, axis=1, keepdims=True)
    zb = z.astype(jnp.bfloat16)
    cbt2 = cbt2_ref[...]
    b2 = b2_ref[...]

    best = jnp.full((t, _W), jnp.inf, jnp.float32)
    bidx = jnp.zeros((t, _W), jnp.int32)
    for c in range(k // _W):
        ab2_c = lax.dot_general(zb, cbt2[:, c * _W:(c + 1) * _W],
                                (((1,), (0,)), ((), ())),
                                preferred_element_type=jnp.float32)
        d_c = (a2 - ab2_c) + b2[:, c * _W:(c + 1) * _W]
        upd = d_c < best
        bidx = jnp.where(upd, jnp.int32(c), bidx)
        best = jnp.where(upd, d_c, best)

    md = jnp.min(best, axis=1)
    lane = lax.broadcasted_iota(jnp.int32, (t, _W), 1)
    cand = bidx * _W + lane
    idx = jnp.min(jnp.where(best == md[:, None], cand, jnp.int32(2**30)),
                  axis=1)
    idx_ref[0, 0, :] = idx

    @pl.when(i == 0)
    def _():
        loss_ref[...] = jnp.zeros((1, 1), jnp.float32)

    loss_ref[...] += (jnp.sum(md) * scale).reshape(1, 1)


def _dist_argmin(z_flat, cbt2, b2):
    n, d = z_flat.shape
    k = cbt2.shape[1]
    grid = (n // _TILE_N,)
    scale = 1.25 / float(n * d)
    return pl.pallas_call(
        functools.partial(_dist_argmin_body, scale),
        grid=grid,
        in_specs=[
            pl.BlockSpec((_TILE_N, d), lambda i: (i, 0)),
            pl.BlockSpec((d, k), lambda i: (0, 0)),
            pl.BlockSpec((1, k), lambda i: (0, 0)),
        ],
        out_specs=[
            pl.BlockSpec((1, 1, _TILE_N), lambda i: (i, 0, 0)),
            pl.BlockSpec((1, 1), lambda i: (0, 0)),
        ],
        out_shape=[
            jax.ShapeDtypeStruct((n // _TILE_N, 1, _TILE_N), jnp.int32),
            jax.ShapeDtypeStruct((1, 1), jnp.float32),
        ],
    )(z_flat, cbt2, b2)


def _sc_gather(codebook, idx):
    b = idx.shape[0]
    d = codebook.shape[1]
    bpw = b // _NW
    mesh = plsc.VectorSubcoreMesh(core_axis_name="c", subcore_axis_name="s")

    @functools.partial(
        pl.kernel,
        mesh=mesh,
        compiler_params=pltpu.CompilerParams(use_tc_tiling_on_sc=False),
        out_type=jax.ShapeDtypeStruct((b, d), jnp.float32),
        scratch_types=[
            pltpu.VMEM((bpw,), jnp.int32),
            pltpu.VMEM((bpw, d), jnp.float32),
            pltpu.SemaphoreType.DMA,
        ],
    )
    def gather_kernel(cb_hbm, idx_hbm, out_hbm, idx_v, rows_v, sem):
        wid = lax.axis_index("s") * _NC + lax.axis_index("c")
        base = wid * bpw
        pltpu.sync_copy(idx_hbm.at[pl.ds(base, bpw)], idx_v)
        pltpu.async_copy(cb_hbm.at[idx_v], rows_v, sem).wait()
        pltpu.sync_copy(rows_v, out_hbm.at[pl.ds(base, bpw)])

    return gather_kernel(codebook, idx)


def kernel(z, codebook):
    d = z.shape[-1]
    z_flat = z.reshape(-1, d)
    cbt = codebook.T
    cbt2 = (cbt + cbt).astype(jnp.bfloat16)
    b2 = jnp.sum(cbt ** 2, axis=0, keepdims=True)
    idx3, loss = _dist_argmin(z_flat, cbt2, b2)
    idx = idx3.reshape(-1)
    q = _sc_gather(codebook, idx)
    return q.reshape(z.shape), loss[0, 0]


# speedup vs baseline: 1.1412x; 1.1412x over previous
import functools

import jax
import jax.numpy as jnp
from jax import lax
from jax.experimental import pallas as pl
from jax.experimental.pallas import tpu as pltpu
from jax.experimental.pallas import tpu_sc as plsc

_TILE_N = 128
_W = 128
_NC, _NS = 2, 16
_NW = _NC * _NS


def _dist_argmin_body(scale, nsteps, z_ref, cbt2_ref, b2_ref,
                      idx_ref, loss_ref, best_s, bidx_s, lacc_s):
    i = pl.program_id(0)
    t = z_ref.shape[0]
    k = cbt2_ref.shape[1]

    bestp = best_s[...]
    bidxp = bidx_s[...]
    md = jnp.min(bestp, axis=1)
    lane = lax.broadcasted_iota(jnp.int32, (t, _W), 1)
    cand = bidxp * _W + lane
    idx = jnp.min(jnp.where(bestp == md[:, None], cand, jnp.int32(2**30)),
                  axis=1)
    idx_ref[0, 0, :] = idx
    lbase = jnp.where(i == 0, 0.0, lacc_s[...])
    lacc_s[...] = lbase + jnp.where(i > 0, md.reshape(t, 1), 0.0)

    z = z_ref[...]
    a2 = jnp.sum(z * z, axis=1, keepdims=True)
    zb = z.astype(jnp.bfloat16)
    cbt2 = cbt2_ref[...]
    b2 = b2_ref[...]
    best = jnp.full((t, _W), jnp.inf, jnp.float32)
    bidx = jnp.zeros((t, _W), jnp.int32)
    for c in range(k // _W):
        ab2_c = lax.dot_general(zb, cbt2[:, c * _W:(c + 1) * _W],
                                (((1,), (0,)), ((), ())),
                                preferred_element_type=jnp.float32)
        d_c = (a2 - ab2_c) + b2[:, c * _W:(c + 1) * _W]
        upd = d_c < best
        bidx = jnp.where(upd, jnp.int32(c), bidx)
        best = jnp.where(upd, d_c, best)
    best_s[...] = best
    bidx_s[...] = bidx

    @pl.when(i == nsteps)
    def _():
        loss_ref[...] = (jnp.sum(lacc_s[...]) * scale).reshape(1, 1)


def _dist_argmin(z_flat, cbt2, b2):
    n, d = z_flat.shape
    k = cbt2.shape[1]
    nsteps = n // _TILE_N
    scale = 1.25 / float(n * d)
    return pl.pallas_call(
        functools.partial(_dist_argmin_body, scale, nsteps),
        grid=(nsteps + 1,),
        in_specs=[
            pl.BlockSpec((_TILE_N, d), lambda i: (jnp.minimum(i, 127), 0)),
            pl.BlockSpec((d, k), lambda i: (0, 0)),
            pl.BlockSpec((1, k), lambda i: (0, 0)),
        ],
        out_specs=[
            pl.BlockSpec((1, 1, _TILE_N),
                         lambda i: (jnp.maximum(i - 1, 0), 0, 0)),
            pl.BlockSpec((1, 1), lambda i: (0, 0)),
        ],
        out_shape=[
            jax.ShapeDtypeStruct((nsteps, 1, _TILE_N), jnp.int32),
            jax.ShapeDtypeStruct((1, 1), jnp.float32),
        ],
        scratch_shapes=[
            pltpu.VMEM((_TILE_N, _W), jnp.float32),
            pltpu.VMEM((_TILE_N, _W), jnp.int32),
            pltpu.VMEM((_TILE_N, 1), jnp.float32),
        ],
    )(z_flat, cbt2, b2)


def _sc_gather(codebook, idx):
    b = idx.shape[0]
    d = codebook.shape[1]
    bpw = b // _NW
    mesh = plsc.VectorSubcoreMesh(core_axis_name="c", subcore_axis_name="s")

    @functools.partial(
        pl.kernel,
        mesh=mesh,
        compiler_params=pltpu.CompilerParams(use_tc_tiling_on_sc=False),
        out_type=jax.ShapeDtypeStruct((b, d), jnp.float32),
        scratch_types=[
            pltpu.VMEM((bpw,), jnp.int32),
            pltpu.VMEM((bpw, d), jnp.float32),
            pltpu.SemaphoreType.DMA,
        ],
    )
    def gather_kernel(cb_hbm, idx_hbm, out_hbm, idx_v, rows_v, sem):
        wid = lax.axis_index("s") * _NC + lax.axis_index("c")
        base = wid * bpw
        pltpu.sync_copy(idx_hbm.at[pl.ds(base, bpw)], idx_v)
        pltpu.async_copy(cb_hbm.at[idx_v], rows_v, sem).wait()
        pltpu.sync_copy(rows_v, out_hbm.at[pl.ds(base, bpw)])

    return gather_kernel(codebook, idx)


def kernel(z, codebook):
    d = z.shape[-1]
    z_flat = z.reshape(-1, d)
    cbt = codebook.T
    cbt2 = (cbt + cbt).astype(jnp.bfloat16)
    b2 = jnp.sum(cbt ** 2, axis=0, keepdims=True)
    idx3, loss = _dist_argmin(z_flat, cbt2, b2)
    idx = idx3.reshape(-1)
    q = _sc_gather(codebook, idx)
    return q.reshape(z.shape), loss[0, 0]


# speedup vs baseline: 1.1991x; 1.0507x over previous
import functools

import jax
import jax.numpy as jnp
from jax import lax
from jax.experimental import pallas as pl
from jax.experimental.pallas import tpu as pltpu
from jax.experimental.pallas import tpu_sc as plsc

_TILE_N = 128
_W = 128
_NC, _NS = 2, 16
_NW = _NC * _NS


def _dist_argmin_body(scale, nsteps, z_ref, cbt2_ref, b2_ref,
                      idx_ref, loss_ref, best_s, bidx_s, lacc_s):
    i = pl.program_id(0)
    t = z_ref.shape[0]
    k = cbt2_ref.shape[1]

    bestp = best_s[...]
    bidxp = bidx_s[...]
    md = jnp.min(bestp, axis=1)
    lane = lax.broadcasted_iota(jnp.int32, (t, _W), 1)
    cand = bidxp * _W + lane
    idx = jnp.min(jnp.where(bestp == md[:, None], cand, jnp.int32(2**30)),
                  axis=1)
    idx_ref[0, 0, :] = idx
    lbase = jnp.where(i == 0, 0.0, lacc_s[...])
    lacc_s[...] = lbase + jnp.where(i > 0, md.reshape(t, 1), 0.0)

    z = z_ref[...]
    a2 = jnp.sum(z * z, axis=1, keepdims=True)
    zb = z.astype(jnp.bfloat16)
    cbt2 = cbt2_ref[...]
    b2 = b2_ref[...]
    ab2 = lax.dot_general(zb, cbt2, (((1,), (0,)), ((), ())),
                          preferred_element_type=jnp.float32)
    best = jnp.full((t, _W), jnp.inf, jnp.float32)
    bidx = jnp.zeros((t, _W), jnp.int32)
    for c in range(k // _W):
        d_c = (a2 - ab2[:, c * _W:(c + 1) * _W]) + b2[:, c * _W:(c + 1) * _W]
        upd = d_c < best
        bidx = jnp.where(upd, jnp.int32(c), bidx)
        best = jnp.where(upd, d_c, best)
    best_s[...] = best
    bidx_s[...] = bidx

    @pl.when(i == nsteps)
    def _():
        loss_ref[...] = (jnp.sum(lacc_s[...]) * scale).reshape(1, 1)


def _dist_argmin(z_flat, cbt2, b2):
    n, d = z_flat.shape
    k = cbt2.shape[1]
    nsteps = n // _TILE_N
    scale = 1.25 / float(n * d)
    return pl.pallas_call(
        functools.partial(_dist_argmin_body, scale, nsteps),
        grid=(nsteps + 1,),
        in_specs=[
            pl.BlockSpec((_TILE_N, d),
                         lambda i: (jnp.minimum(i, nsteps - 1), 0)),
            pl.BlockSpec((d, k), lambda i: (0, 0)),
            pl.BlockSpec((1, k), lambda i: (0, 0)),
        ],
        out_specs=[
            pl.BlockSpec((1, 1, _TILE_N),
                         lambda i: (jnp.maximum(i - 1, 0), 0, 0)),
            pl.BlockSpec((1, 1), lambda i: (0, 0)),
        ],
        out_shape=[
            jax.ShapeDtypeStruct((nsteps, 1, _TILE_N), jnp.int32),
            jax.ShapeDtypeStruct((1, 1), jnp.float32),
        ],
        scratch_shapes=[
            pltpu.VMEM((_TILE_N, _W), jnp.float32),
            pltpu.VMEM((_TILE_N, _W), jnp.int32),
            pltpu.VMEM((_TILE_N, 1), jnp.float32),
        ],
    )(z_flat, cbt2, b2)


def _sc_gather(codebook, idx):
    b = idx.shape[0]
    d = codebook.shape[1]
    bpw = b // _NW
    mesh = plsc.VectorSubcoreMesh(core_axis_name="c", subcore_axis_name="s")

    @functools.partial(
        pl.kernel,
        mesh=mesh,
        compiler_params=pltpu.CompilerParams(use_tc_tiling_on_sc=False),
        out_type=jax.ShapeDtypeStruct((b, d), jnp.float32),
        scratch_types=[
            pltpu.VMEM((bpw,), jnp.int32),
            pltpu.VMEM((bpw, d), jnp.float32),
            pltpu.SemaphoreType.DMA,
        ],
    )
    def gather_kernel(cb_hbm, idx_hbm, out_hbm, idx_v, rows_v, sem):
        wid = lax.axis_index("s") * _NC + lax.axis_index("c")
        base = wid * bpw
        pltpu.sync_copy(idx_hbm.at[pl.ds(base, bpw)], idx_v)
        pltpu.async_copy(cb_hbm.at[idx_v], rows_v, sem).wait()
        pltpu.sync_copy(rows_v, out_hbm.at[pl.ds(base, bpw)])

    return gather_kernel(codebook, idx)


def kernel(z, codebook):
    d = z.shape[-1]
    z_flat = z.reshape(-1, d)
    cbt = codebook.T
    cbt2 = (cbt + cbt).astype(jnp.bfloat16)
    b2 = jnp.sum(cbt ** 2, axis=0, keepdims=True)
    idx3, loss = _dist_argmin(z_flat, cbt2, b2)
    idx = idx3.reshape(-1)
    q = _sc_gather(codebook, idx)
    return q.reshape(z.shape), loss[0, 0]
